# Initial kernel scaffold; baseline (speedup 1.0000x reference)
import functools

import jax
import jax.numpy as jnp
from jax import lax
from jax.experimental import pallas as pl
from jax.experimental.pallas import tpu as pltpu
from jax.experimental.pallas import tpu_sc as plsc

_N_EDGES = 320000
_D = 128
_N_SEG = 10000
_NC = 2
_NS = 16
_EDGES_PER_TILE = _N_EDGES // (_NC * _NS)
_CHUNK = 80
_N_CHUNKS = _EDGES_PER_TILE // _CHUNK
_SEG_PER_TILE = _N_SEG // _NS
_ZCHUNK = 125


def _sc_body(x_hbm, idx_hbm, out_hbm, acc_sh, rows_v, idx_v, zeros_v):
    c = lax.axis_index("c")
    s = lax.axis_index("s")
    tid = c * _NS + s

    zvec = jnp.zeros((16,), jnp.float32)
    def _zero_store(i, _):
        zeros_v[pl.ds(i * 16, 16)] = zvec
        return ()
    lax.fori_loop(0, (_ZCHUNK * _D) // 16, _zero_store, (), unroll=8)
    zeros2d = zeros_v.reshape(_ZCHUNK, _D)
    def _zero_copy(j, _):
        pltpu.sync_copy(zeros2d, acc_sh.at[pl.ds(s * _SEG_PER_TILE + j * _ZCHUNK, _ZCHUNK)])
        return ()
    lax.fori_loop(0, _SEG_PER_TILE // _ZCHUNK, _zero_copy, ())
    plsc.subcore_barrier()

    base = tid * _EDGES_PER_TILE
    def _chunk(i, _):
        off = base + i * _CHUNK
        pltpu.sync_copy(idx_hbm.at[pl.ds(off, _CHUNK)], idx_v)
        pltpu.sync_copy(x_hbm.at[pl.ds(off, _CHUNK)], rows_v)
        pltpu.sync_copy(rows_v, acc_sh.at[idx_v], add=True)
        return ()
    lax.fori_loop(0, _N_CHUNKS, _chunk, ())
    plsc.subcore_barrier()

    def _wb(j, _):
        r0 = s * _SEG_PER_TILE + j * _ZCHUNK
        pltpu.sync_copy(acc_sh.at[pl.ds(r0, _ZCHUNK)], out_hbm.at[c, pl.ds(r0, _ZCHUNK)])
        return ()
    lax.fori_loop(0, _SEG_PER_TILE // _ZCHUNK, _wb, ())


def _tc_add(a_ref, b_ref, o_ref):
    o_ref[...] = a_ref[0] + b_ref[0]


@jax.jit
def kernel(x, index):
    mesh = plsc.VectorSubcoreMesh(core_axis_name="c", subcore_axis_name="s")
    partials = pl.kernel(
        _sc_body,
        out_type=jax.ShapeDtypeStruct((_NC, _N_SEG, _D), jnp.float32),
        mesh=mesh,
        scratch_types=[
            pltpu.VMEM_SHARED((_N_SEG, _D), jnp.float32),
            pltpu.VMEM((_CHUNK, _D), jnp.float32),
            pltpu.VMEM((_CHUNK,), jnp.int32),
            pltpu.VMEM((_ZCHUNK * _D,), jnp.float32),
        ],
    )(x, index.astype(jnp.int32))

    blk = 1250
    out = pl.pallas_call(
        _tc_add,
        grid=(_N_SEG // blk,),
        in_specs=[
            pl.BlockSpec((1, blk, _D), lambda i: (0, i, 0)),
            pl.BlockSpec((1, blk, _D), lambda i: (1, i, 0)),
        ],
        out_specs=pl.BlockSpec((blk, _D), lambda i: (i, 0)),
        out_shape=jax.ShapeDtypeStruct((_N_SEG, _D), jnp.float32),
    )(partials, partials)
    return out


# speedup vs baseline: 3.5643x; 3.5643x over previous
import functools

import jax
import jax.numpy as jnp
from jax import lax
from jax.experimental import pallas as pl
from jax.experimental.pallas import tpu as pltpu
from jax.experimental.pallas import tpu_sc as plsc

_N_EDGES = 320000
_D = 128
_N_SEG = 10000
_NC = 2
_NS = 16
_EDGES_PER_TILE = _N_EDGES // (_NC * _NS)
_CHUNK = 80
_N_CHUNKS = _EDGES_PER_TILE // _CHUNK
_SEG_PER_TILE = _N_SEG // _NS
_ZCHUNK = 125
_WB_CHUNK = 16
_N_WB_CHUNKS = _N_SEG // _WB_CHUNK


def _sc_body(x_hbm, idx_hbm, out_hbm, acc_sh, rows_v, idx_v, zeros_v):
    c = lax.axis_index("c")
    s = lax.axis_index("s")
    tid = c * _NS + s

    zvec = jnp.zeros((16,), jnp.float32)
    def _zero_row(i, _):
        def _zero_lane(k, _):
            zeros_v[i, pl.ds(k * 16, 16)] = zvec
            return ()
        lax.fori_loop(0, _D // 16, _zero_lane, (), unroll=True)
        return ()
    lax.fori_loop(0, _ZCHUNK, _zero_row, ())
    def _zero_copy(j, _):
        pltpu.sync_copy(zeros_v, acc_sh.at[pl.ds(s * _SEG_PER_TILE + j * _ZCHUNK, _ZCHUNK)])
        return ()
    lax.fori_loop(0, _SEG_PER_TILE // _ZCHUNK, _zero_copy, ())
    plsc.subcore_barrier()

    base = tid * _EDGES_PER_TILE
    def _chunk(i, _):
        off = pl.multiple_of(base + i * _CHUNK, 8)
        pltpu.sync_copy(idx_hbm.at[pl.ds(off, _CHUNK)], idx_v)
        pltpu.sync_copy(x_hbm.at[pl.ds(off, _CHUNK)], rows_v)
        pltpu.sync_copy(rows_v, acc_sh.at[idx_v], add=True)
        return ()
    lax.fori_loop(0, _N_CHUNKS, _chunk, ())
    plsc.subcore_barrier()

    def _wb(j, _):
        cidx = j * _NS + s
        @pl.when(cidx < _N_WB_CHUNKS)
        def _():
            r0 = pl.multiple_of(cidx * _WB_CHUNK, 16)
            pltpu.sync_copy(acc_sh.at[pl.ds(r0, _WB_CHUNK)],
                            out_hbm.at[c, pl.ds(r0, _WB_CHUNK)])
        return ()
    lax.fori_loop(0, (_N_WB_CHUNKS + _NS - 1) // _NS, _wb, ())


def _tc_add(a_ref, b_ref, o_ref):
    o_ref[...] = a_ref[0] + b_ref[0]


@jax.jit
def kernel(x, index):
    mesh = plsc.VectorSubcoreMesh(core_axis_name="c", subcore_axis_name="s")
    partials = pl.kernel(
        _sc_body,
        out_type=jax.ShapeDtypeStruct((_NC, _N_SEG, _D), jnp.float32),
        mesh=mesh,
        scratch_types=[
            pltpu.VMEM_SHARED((_N_SEG, _D), jnp.float32),
            pltpu.VMEM((_CHUNK, _D), jnp.float32),
            pltpu.VMEM((_CHUNK,), jnp.int32),
            pltpu.VMEM((_ZCHUNK, _D), jnp.float32),
        ],
    )(x, index.astype(jnp.int32))

    blk = 2000
    out = pl.pallas_call(
        _tc_add,
        grid=(_N_SEG // blk,),
        in_specs=[
            pl.BlockSpec((1, blk, _D), lambda i: (0, i, 0)),
            pl.BlockSpec((1, blk, _D), lambda i: (1, i, 0)),
        ],
        out_specs=pl.BlockSpec((blk, _D), lambda i: (i, 0)),
        out_shape=jax.ShapeDtypeStruct((_N_SEG, _D), jnp.float32),
    )(partials, partials)
    return out


# speedup vs baseline: 7.0741x; 1.9847x over previous
import functools

import jax
import jax.numpy as jnp
from jax import lax
from jax.experimental import pallas as pl
from jax.experimental.pallas import tpu as pltpu
from jax.experimental.pallas import tpu_sc as plsc

_N_EDGES = 320000
_D = 128
_N_SEG = 10000
_NC = 2
_NS = 16
_EDGES_PER_TILE = _N_EDGES // (_NC * _NS)
_CHUNK = 80
_N_CHUNKS = _EDGES_PER_TILE // _CHUNK
_NBUF = 3
_ZROWS = 5
_SEG_PER_TILE = _N_SEG // _NS
_ZCHUNK = 125
_WB_CHUNK = 16
_N_WB_CHUNKS = _N_SEG // _WB_CHUNK


def _sc_body(x_hbm, idx3_hbm, out_hbm, acc_sh, idx_v, rows_v, zeros_v,
             idx_sem, *row_sems):
    c = lax.axis_index("c")
    s = lax.axis_index("s")
    tid = c * _NS + s
    base = tid * _EDGES_PER_TILE

    def _row_src(ci):
        off = pl.multiple_of(base + ci * _CHUNK, 16)
        return x_hbm.at[pl.ds(off, _CHUNK)]

    pltpu.async_copy(idx3_hbm.at[tid], idx_v, idx_sem)
    for b in range(_NBUF):
        pltpu.async_copy(_row_src(b), rows_v.at[b], row_sems[b])

    zvec = jnp.zeros((16,), jnp.float32)
    def _zero_row(i, _):
        def _zero_lane(k, _):
            zeros_v[i, pl.ds(k * 16, 16)] = zvec
            return ()
        lax.fori_loop(0, _D // 16, _zero_lane, (), unroll=True)
        return ()
    lax.fori_loop(0, _ZROWS, _zero_row, ())
    def _zero_copy(j, _):
        pltpu.sync_copy(zeros_v,
                        acc_sh.at[pl.ds(s * _SEG_PER_TILE + j * _ZROWS, _ZROWS)])
        return ()
    lax.fori_loop(0, _SEG_PER_TILE // _ZROWS, _zero_copy, ())
    plsc.subcore_barrier()
    pltpu.make_async_copy(idx3_hbm.at[tid], idx_v, idx_sem).wait()

    n_outer = (_N_CHUNKS + _NBUF - 1) // _NBUF
    def _outer(j0, _):
        for b in range(_NBUF):
            ci = j0 * _NBUF + b
            @pl.when(ci < _N_CHUNKS)
            def _():
                pltpu.make_async_copy(_row_src(ci), rows_v.at[b],
                                      row_sems[b]).wait()
                pltpu.sync_copy(rows_v.at[b], acc_sh.at[idx_v.at[ci]],
                                add=True)
                @pl.when(ci + _NBUF < _N_CHUNKS)
                def _():
                    pltpu.async_copy(_row_src(ci + _NBUF), rows_v.at[b],
                                     row_sems[b])
        return ()
    lax.fori_loop(0, n_outer, _outer, ())
    plsc.subcore_barrier()

    def _wb(j, _):
        cw = j * _NS + s
        @pl.when(cw < _N_WB_CHUNKS)
        def _():
            r0 = pl.multiple_of(cw * _WB_CHUNK, 16)
            pltpu.sync_copy(acc_sh.at[pl.ds(r0, _WB_CHUNK)],
                            out_hbm.at[c, pl.ds(r0, _WB_CHUNK)])
        return ()
    lax.fori_loop(0, (_N_WB_CHUNKS + _NS - 1) // _NS, _wb, ())


def _tc_add(a_ref, b_ref, o_ref):
    o_ref[...] = a_ref[0] + b_ref[0]


@jax.jit
def kernel(x, index):
    idx3 = index.astype(jnp.int32).reshape(_NC * _NS, _N_CHUNKS, _CHUNK)
    mesh = plsc.VectorSubcoreMesh(core_axis_name="c", subcore_axis_name="s")
    partials = pl.kernel(
        _sc_body,
        out_type=jax.ShapeDtypeStruct((_NC, _N_SEG, _D), jnp.float32),
        mesh=mesh,
        scratch_types=[
            pltpu.VMEM_SHARED((_N_SEG, _D), jnp.float32),
            pltpu.VMEM((_N_CHUNKS, _CHUNK), jnp.int32),
            pltpu.VMEM((_NBUF, _CHUNK, _D), jnp.float32),
            pltpu.VMEM((_ZROWS, _D), jnp.float32),
            pltpu.SemaphoreType.DMA,
            *([pltpu.SemaphoreType.DMA] * _NBUF),
        ],
    )(x, idx3)

    blk = 2000
    out = pl.pallas_call(
        _tc_add,
        grid=(_N_SEG // blk,),
        in_specs=[
            pl.BlockSpec((1, blk, _D), lambda i: (0, i, 0)),
            pl.BlockSpec((1, blk, _D), lambda i: (1, i, 0)),
        ],
        out_specs=pl.BlockSpec((blk, _D), lambda i: (i, 0)),
        out_shape=jax.ShapeDtypeStruct((_N_SEG, _D), jnp.float32),
    )(partials, partials)
    return out
